# baseline (speedup 1.0000x reference)
import jax
import jax.numpy as jnp
from jax.experimental import pallas as pl
from jax.experimental.pallas import tpu as pltpu

_TM = 1024
_KE = 128


def _fused_body(idx_ref, x_ref, a_ref, w_hbm_ref, wext_hbm_ref, o_ref,
                xs_ref, wcat_ref, sem_w, sem_e):
    bi = pl.program_id(0)
    mi = pl.program_id(1)
    din = x_ref.shape[2]
    r = 16

    @pl.when((bi == 0) & (mi == 0))
    def _():
        cp = pltpu.make_async_copy(w_hbm_ref, wcat_ref.at[:, :din], sem_w)
        cp.start()
        cp.wait()

    @pl.when(mi == 0)
    def _():
        cp = pltpu.make_async_copy(
            wext_hbm_ref.at[idx_ref[bi]], wcat_ref.at[:, din:], sem_e)
        cp.start()
        cp.wait()

    xb = x_ref[0].astype(jnp.bfloat16)
    xs_ref[:, :din] = xb
    inter = jax.lax.dot_general(
        xb, a_ref[0], (((1,), (1,)), ((), ())),
        preferred_element_type=jnp.float32)
    one = (jax.lax.broadcasted_iota(jnp.int32, inter.shape, 1) == r)
    xs_ref[:, din:] = (inter + one.astype(jnp.float32)).astype(jnp.bfloat16)
    o_ref[0] = jax.lax.dot_general(
        xs_ref[...], wcat_ref[...], (((1,), (1,)), ((), ())),
        preferred_element_type=jnp.float32)


def kernel(x, adapter_indices, W, b, lora_a, lora_b):
    B, S, DIN = x.shape
    DOUT = W.shape[0]
    E, R, _ = lora_a.shape
    idx = adapter_indices.astype(jnp.int32)
    w_bf = W.astype(jnp.bfloat16)
    a_ext = jnp.concatenate(
        [lora_a.astype(jnp.bfloat16),
         jnp.zeros((E, _KE - R, DIN), jnp.bfloat16)], axis=1)
    w_ext = jnp.concatenate(
        [lora_b.astype(jnp.bfloat16),
         jnp.broadcast_to(b[None, :, None], (E, DOUT, 1)).astype(jnp.bfloat16),
         jnp.zeros((E, DOUT, _KE - R - 1), jnp.bfloat16)], axis=2)

    grid = (B, S // _TM)

    grid_spec = pltpu.PrefetchScalarGridSpec(
        num_scalar_prefetch=1,
        grid=grid,
        in_specs=[
            pl.BlockSpec((1, _TM, DIN), lambda bi, mi, idx_ref: (bi, mi, 0)),
            pl.BlockSpec((1, _KE, DIN), lambda bi, mi, idx_ref: (idx_ref[bi], 0, 0)),
            pl.BlockSpec(memory_space=pltpu.MemorySpace.HBM),
            pl.BlockSpec(memory_space=pltpu.MemorySpace.HBM),
        ],
        out_specs=pl.BlockSpec((1, _TM, DOUT), lambda bi, mi, idx_ref: (bi, mi, 0)),
        scratch_shapes=[
            pltpu.VMEM((_TM, DIN + _KE), jnp.bfloat16),
            pltpu.VMEM((DOUT, DIN + _KE), jnp.bfloat16),
            pltpu.SemaphoreType.DMA,
            pltpu.SemaphoreType.DMA,
        ],
    )

    return pl.pallas_call(
        _fused_body,
        grid_spec=grid_spec,
        out_shape=jax.ShapeDtypeStruct((B, S, DOUT), jnp.float32),
    )(idx, x, a_ext, w_bf, w_ext)


# speedup vs baseline: 1.3281x; 1.3281x over previous
import jax
import jax.numpy as jnp
from jax.experimental import pallas as pl
from jax.experimental.pallas import tpu as pltpu

_TM = 1024
_TN = 512


def _fused_body(idx_ref, x_ref, w_ref, bias_ref, a_ref, bb_ref, o_ref):
    x = x_ref[0].astype(jnp.bfloat16)
    a = a_ref[0].astype(jnp.bfloat16)
    inter = jax.lax.dot_general(
        x, a, (((1,), (1,)), ((), ())),
        preferred_element_type=jnp.float32)
    ib = inter.astype(jnp.bfloat16)
    bb = bb_ref[0].astype(jnp.bfloat16)
    dout = bb.shape[0]
    for n in range(0, dout, _TN):
        acc = jax.lax.dot_general(
            x, w_ref[n:n + _TN, :], (((1,), (1,)), ((), ())),
            preferred_element_type=jnp.float32)
        lora = jax.lax.dot_general(
            ib, bb[n:n + _TN, :], (((1,), (1,)), ((), ())),
            preferred_element_type=jnp.float32)
        o_ref[0, :, n:n + _TN] = acc + lora + bias_ref[:, n:n + _TN]


def kernel(x, adapter_indices, W, b, lora_a, lora_b):
    B, S, DIN = x.shape
    DOUT = W.shape[0]
    E, R, _ = lora_a.shape
    idx = adapter_indices.astype(jnp.int32)
    bias = b.reshape(1, DOUT)
    w_bf = W.astype(jnp.bfloat16)

    grid = (B, S // _TM)

    grid_spec = pltpu.PrefetchScalarGridSpec(
        num_scalar_prefetch=1,
        grid=grid,
        in_specs=[
            pl.BlockSpec((1, _TM, DIN), lambda bi, mi, idx_ref: (bi, mi, 0)),
            pl.BlockSpec((DOUT, DIN), lambda bi, mi, idx_ref: (0, 0)),
            pl.BlockSpec((1, DOUT), lambda bi, mi, idx_ref: (0, 0)),
            pl.BlockSpec((1, R, DIN), lambda bi, mi, idx_ref: (idx_ref[bi], 0, 0)),
            pl.BlockSpec((1, DOUT, R), lambda bi, mi, idx_ref: (idx_ref[bi], 0, 0)),
        ],
        out_specs=pl.BlockSpec((1, _TM, DOUT), lambda bi, mi, idx_ref: (bi, mi, 0)),
    )

    return pl.pallas_call(
        _fused_body,
        grid_spec=grid_spec,
        out_shape=jax.ShapeDtypeStruct((B, S, DOUT), jnp.float32),
    )(idx, x, w_bf, bias, lora_a, lora_b)


# speedup vs baseline: 1.3794x; 1.0387x over previous
import jax
import jax.numpy as jnp
from jax.experimental import pallas as pl
from jax.experimental.pallas import tpu as pltpu

_TM = 512
_TN = 512


def _fused_body(idx_ref, x_ref, w_ref, bias_ref, a_ref, bb_ref, o_ref, wb_ref):
    bi = pl.program_id(0)
    mi = pl.program_id(1)

    @pl.when((bi == 0) & (mi == 0))
    def _():
        wb_ref[...] = w_ref[...].astype(jnp.bfloat16)

    x = x_ref[0].astype(jnp.bfloat16)
    a = a_ref[0].astype(jnp.bfloat16)
    inter = jax.lax.dot_general(
        x, a, (((1,), (1,)), ((), ())),
        preferred_element_type=jnp.float32)
    ib = inter.astype(jnp.bfloat16)
    bb = bb_ref[0].astype(jnp.bfloat16)
    dout = bb.shape[0]
    for n in range(0, dout, _TN):
        acc = jax.lax.dot_general(
            x, wb_ref[n:n + _TN, :], (((1,), (1,)), ((), ())),
            preferred_element_type=jnp.float32)
        lora = jax.lax.dot_general(
            ib, bb[n:n + _TN, :], (((1,), (1,)), ((), ())),
            preferred_element_type=jnp.float32)
        o_ref[0, :, n:n + _TN] = acc + lora + bias_ref[:, n:n + _TN]


def kernel(x, adapter_indices, W, b, lora_a, lora_b):
    B, S, DIN = x.shape
    DOUT = W.shape[0]
    E, R, _ = lora_a.shape
    idx = adapter_indices.astype(jnp.int32)
    bias = b.reshape(1, DOUT)

    grid = (B, S // _TM)

    grid_spec = pltpu.PrefetchScalarGridSpec(
        num_scalar_prefetch=1,
        grid=grid,
        in_specs=[
            pl.BlockSpec((1, _TM, DIN), lambda bi, mi, idx_ref: (bi, mi, 0)),
            pl.BlockSpec((DOUT, DIN), lambda bi, mi, idx_ref: (0, 0)),
            pl.BlockSpec((1, DOUT), lambda bi, mi, idx_ref: (0, 0)),
            pl.BlockSpec((1, R, DIN), lambda bi, mi, idx_ref: (idx_ref[bi], 0, 0)),
            pl.BlockSpec((1, DOUT, R), lambda bi, mi, idx_ref: (idx_ref[bi], 0, 0)),
        ],
        out_specs=pl.BlockSpec((1, _TM, DOUT), lambda bi, mi, idx_ref: (bi, mi, 0)),
        scratch_shapes=[pltpu.VMEM((DOUT, DIN), jnp.bfloat16)],
    )

    return pl.pallas_call(
        _fused_body,
        grid_spec=grid_spec,
        out_shape=jax.ShapeDtypeStruct((B, S, DOUT), jnp.float32),
    )(idx, x, W, bias, lora_a, lora_b)
